# baseline (speedup 1.0000x reference)
import jax
import jax.numpy as jnp
import numpy as np
from jax import lax
from jax.experimental import pallas as pl
from jax.experimental.pallas import tpu as pltpu
from jax.experimental.pallas import tpu_sc as plsc

_R1SQ = np.float32(0.2 * 0.2)
_R2SQ = np.float32(0.4 * 0.4)
_MAXNB = 64
_NBUF = 80
_NUM_GRAPHS = 8
_NEG_INF = np.float32(-np.inf)


def _make_select_kernel(n_points):
    info = plsc.get_sparse_core_info()
    nc, ns = info.num_cores, info.num_subcores
    nw = nc * ns
    assert n_points % nw == 0
    per_w = n_points // nw
    grp = 16
    assert per_w % grp == 0

    mesh = plsc.VectorSubcoreMesh(core_axis_name="c", subcore_axis_name="s")

    out_type = (
        [jax.ShapeDtypeStruct((n_points * _NBUF,), jnp.float32) for _ in range(6)]
        + [jax.ShapeDtypeStruct((n_points * _NBUF,), jnp.int32)]
        + [jax.ShapeDtypeStruct((n_points,), jnp.int32) for _ in range(2)]
    )

    scratch = (
        [pltpu.VMEM((n_points,), jnp.float32) for _ in range(7)]
        + [pltpu.VMEM((per_w,), jnp.int32) for _ in range(2)]
        + [pltpu.VMEM((grp * _NBUF,), jnp.float32) for _ in range(6)]
        + [pltpu.VMEM((grp * _NBUF,), jnp.int32)]
        + [pltpu.VMEM((grp,), jnp.int32) for _ in range(2)]
    )

    def body(px_h, py_h, pz_h, sq_h, rs_h, re_h,
             r1x_h, r1y_h, r1z_h, r2x_h, r2y_h, r2z_h, nb2_h, c1_h, c2_h,
             px, py, pz, qx, qy, qz, sq, rs, re,
             s1x, s1y, s1z, s2x, s2y, s2z, snb, sc1, sc2):
        wid = lax.axis_index("c") * ns + lax.axis_index("s")
        base = wid * per_w
        for src, dst in ((px_h, px), (py_h, py), (pz_h, pz), (sq_h, sq)):
            pltpu.sync_copy(src, dst)
        pltpu.sync_copy(rs_h.at[pl.ds(base, per_w)], rs)
        pltpu.sync_copy(re_h.at[pl.ds(base, per_w)], re)

        def bf16_round(v):
            u = plsc.bitcast(v, jnp.uint32)
            rb = (u >> jnp.uint32(16)) & jnp.uint32(1)
            u2 = (u + jnp.uint32(0x7FFF) + rb) & jnp.uint32(0xFFFF0000)
            return plsc.bitcast(u2, jnp.float32)

        def qinit(t, _):
            o = t * 16
            qx[pl.ds(o, 16)] = bf16_round(px[pl.ds(o, 16)])
            qy[pl.ds(o, 16)] = bf16_round(py[pl.ds(o, 16)])
            qz[pl.ds(o, 16)] = bf16_round(pz[pl.ds(o, 16)])
            return 0

        lax.fori_loop(0, n_points // 16, qinit, 0)

        lanes = lax.iota(jnp.int32, 16)
        zeros16 = jnp.zeros((16,), jnp.int32)

        def do_group(gi, _):
            gbase = gi * grp
            xv = px[pl.ds(base + gbase, grp)]
            yv = py[pl.ds(base + gbase, grp)]
            zv = pz[pl.ds(base + gbase, grp)]
            qxg = qx[pl.ds(base + gbase, grp)]
            qyg = qy[pl.ds(base + gbase, grp)]
            qzg = qz[pl.ds(base + gbase, grp)]
            sqg = sq[pl.ds(base + gbase, grp)]
            rsv = rs[pl.ds(gbase, grp)]
            rev = re[pl.ds(gbase, grp)]
            cv1 = jnp.zeros((grp,), jnp.int32)
            cv2 = jnp.zeros((grp,), jnp.int32)

            for ip in range(grp):
                xi, yi, zi = xv[ip], yv[ip], zv[ip]
                qxi, qyi, qzi = qxg[ip], qyg[ip], qzg[ip]
                sqi = sqg[ip]
                s = rsv[ip]
                e = rev[ip]

                for k in range(_NBUF // 16):
                    snb[pl.ds(ip * _NBUF + k * 16, 16)] = zeros16

                def chunk(carry, ip=ip):
                    jb, c1, c2 = carry
                    qxv = qx[pl.ds(jb, 16)]
                    qyv = qy[pl.ds(jb, 16)]
                    qzv = qz[pl.ds(jb, 16)]
                    sqv = sq[pl.ds(jb, 16)]
                    cross = qxi * qxv + qyi * qyv + qzi * qzv
                    d2 = (sqi + sqv) - jnp.float32(2.0) * cross
                    inb = lanes < (e - jb)
                    m1 = (d2 <= _R1SQ) & inb
                    m2 = (d2 <= _R2SQ) & inb
                    rx = px[pl.ds(jb, 16)] - xi
                    ry = py[pl.ds(jb, 16)] - yi
                    rz = pz[pl.ds(jb, 16)] - zi
                    jv = jb + lanes

                    m1s = m1 & (c1 < _MAXNB)
                    m2s = m2 & (c2 < _MAXNB)
                    off1 = ip * _NBUF + jnp.minimum(c1, _NBUF - 16)
                    off2 = ip * _NBUF + jnp.minimum(c2, _NBUF - 16)
                    plsc.store_compressed(s1x.at[pl.ds(off1, 16)], rx, mask=m1s)
                    plsc.store_compressed(s1y.at[pl.ds(off1, 16)], ry, mask=m1s)
                    plsc.store_compressed(s1z.at[pl.ds(off1, 16)], rz, mask=m1s)
                    plsc.store_compressed(s2x.at[pl.ds(off2, 16)], rx, mask=m2s)
                    plsc.store_compressed(s2y.at[pl.ds(off2, 16)], ry, mask=m2s)
                    plsc.store_compressed(s2z.at[pl.ds(off2, 16)], rz, mask=m2s)
                    plsc.store_compressed(snb.at[pl.ds(off2, 16)], jv, mask=m2s)

                    n1 = plsc.all_reduce_population_count(m1)[0]
                    n2 = plsc.all_reduce_population_count(m2)[0]
                    c1 = jnp.where(c1 < _MAXNB, c1 + n1, c1)
                    c2 = jnp.where(c2 < _MAXNB, c2 + n2, c2)
                    return jb + 16, c1, c2

                def cond(carry):
                    jb, _, _ = carry
                    return jb < e

                _, c1f, c2f = lax.while_loop(cond, chunk,
                                             (s, jnp.int32(0), jnp.int32(0)))
                lane_is_ip = lanes == ip
                cv1 = jnp.where(lane_is_ip, jnp.minimum(c1f, _MAXNB), cv1)
                cv2 = jnp.where(lane_is_ip, jnp.minimum(c2f, _MAXNB), cv2)

            sc1[...] = cv1
            sc2[...] = cv2

            row = base + gbase
            for st, hb in ((s1x, r1x_h), (s1y, r1y_h), (s1z, r1z_h),
                           (s2x, r2x_h), (s2y, r2y_h), (s2z, r2z_h),
                           (snb, nb2_h)):
                pltpu.sync_copy(st, hb.at[pl.ds(row * _NBUF, grp * _NBUF)])
            pltpu.sync_copy(sc1, c1_h.at[pl.ds(row, grp)])
            pltpu.sync_copy(sc2, c2_h.at[pl.ds(row, grp)])
            return 0

        lax.fori_loop(0, per_w // grp, do_group, 0)

    return pl.kernel(
        body, out_type=out_type, mesh=mesh,
        compiler_params=pltpu.CompilerParams(needs_layout_passes=False),
        scratch_types=scratch)


def _make_gather_kernel(n_rows, d):
    info = plsc.get_sparse_core_info()
    nw = info.num_cores * info.num_subcores
    assert n_rows % nw == 0
    per_w = n_rows // nw
    chunk = 128
    assert per_w % chunk == 0

    mesh = plsc.VectorSubcoreMesh(core_axis_name="c", subcore_axis_name="s")

    nbuf = 4
    assert per_w % (chunk * nbuf) == 0

    def body(tab_h, idx_h, out_h, idx_v, rows_v, sem, osem):
        wid = lax.axis_index("c") * info.num_subcores + lax.axis_index("s")
        base = wid * per_w
        pltpu.sync_copy(idx_h.at[pl.ds(base, per_w)], idx_v)

        def step(t0, _):
            gs = []
            for k in range(nbuf):
                t = t0 * nbuf + k
                gs.append(pltpu.async_copy(
                    tab_h.at[idx_v.at[pl.ds(t * chunk, chunk)]],
                    rows_v.at[k], sem))
            for h in gs:
                h.wait()
            ws = []
            for k in range(nbuf):
                t = t0 * nbuf + k
                ws.append(pltpu.async_copy(
                    rows_v.at[k],
                    out_h.at[pl.ds(base + t * chunk, chunk), :], osem))
            for h in ws:
                h.wait()
            return 0

        lax.fori_loop(0, per_w // (chunk * nbuf), step, 0)

    return pl.kernel(
        body,
        out_type=jax.ShapeDtypeStruct((n_rows, d), jnp.float32),
        mesh=mesh,
        compiler_params=pltpu.CompilerParams(needs_layout_passes=False),
        scratch_types=[
            pltpu.VMEM((per_w,), jnp.int32),
            pltpu.VMEM((nbuf, chunk, d), jnp.float32),
            pltpu.SemaphoreType.DMA,
            pltpu.SemaphoreType.DMA,
        ],
    )


def _sa1_body(x, vmask, w1, b1, w2, b2, w3, b3, out):
    rows = x.shape[0]
    t = rows // _MAXNB
    m = jnp.dot(x[...], w1[...], preferred_element_type=jnp.float32)
    m = jnp.maximum(m + b1[...], 0.0)
    m = jnp.dot(m, w2[...], preferred_element_type=jnp.float32)
    m = jnp.maximum(m + b2[...], 0.0)
    m = jnp.dot(m, w3[...], preferred_element_type=jnp.float32)
    m = jnp.maximum(m + b3[...], 0.0)
    m = jnp.where(vmask[...] > 0.5, m, _NEG_INF)
    out[...] = jnp.max(m.reshape(t, _MAXNB, m.shape[-1]), axis=1)


def _run_sa1(x1, vmask1, params, t=128):
    nrows = x1.shape[0]
    n = nrows // _MAXNB
    (w1, b1), (w2, b2), (w3, b3) = params
    fo = w3.shape[1]
    full = lambda a: pl.BlockSpec(a.shape, lambda i: tuple(0 for _ in a.shape))
    args = (x1, vmask1, w1, b1.reshape(1, -1), w2, b2.reshape(1, -1),
            w3, b3.reshape(1, -1))
    return pl.pallas_call(
        _sa1_body,
        grid=(n // t,),
        in_specs=[pl.BlockSpec((t * _MAXNB, 3), lambda i: (i, 0)),
                  pl.BlockSpec((t * _MAXNB, 1), lambda i: (i, 0))] +
                 [full(a) for a in args[2:]],
        out_specs=pl.BlockSpec((t, fo), lambda i: (i, 0)),
        out_shape=jax.ShapeDtypeStruct((n, fo), jnp.float32),
    )(*args)


def _sa2_body(h1g, x2, vmask, pos, bat,
              w1a, w1b, b1, w2, b2, w3, b3,
              ga, gb, gbias, g2w, g2b, g3w, g3b,
              h2_out, scene_out, scene_acc):
    t = pos.shape[0]
    i = pl.program_id(0)

    @pl.when(i == 0)
    def _():
        scene_acc[...] = jnp.full_like(scene_acc[...], _NEG_INF)

    m = jnp.dot(h1g[...], w1a[...], preferred_element_type=jnp.float32)
    m = m + jnp.dot(x2[...], w1b[...], preferred_element_type=jnp.float32)
    m = jnp.maximum(m + b1[...], 0.0)
    m = jnp.dot(m, w2[...], preferred_element_type=jnp.float32)
    m = jnp.maximum(m + b2[...], 0.0)
    m = jnp.dot(m, w3[...], preferred_element_type=jnp.float32)
    m = jnp.maximum(m + b3[...], 0.0)
    m = jnp.where(vmask[...] > 0.5, m, _NEG_INF)
    h2 = jnp.max(m.reshape(t, _MAXNB, m.shape[-1]), axis=1)
    h2_out[...] = h2

    g = jnp.dot(h2, ga[...], preferred_element_type=jnp.float32)
    g = g + jnp.dot(pos[...], gb[...], preferred_element_type=jnp.float32)
    g = jnp.maximum(g + gbias[...], 0.0)
    g = jnp.dot(g, g2w[...], preferred_element_type=jnp.float32)
    g = jnp.maximum(g + g2b[...], 0.0)
    g = jnp.dot(g, g3w[...], preferred_element_type=jnp.float32)
    g = jnp.maximum(g + g3b[...], 0.0)

    b = bat[...]
    for s in range(_NUM_GRAPHS):
        gs = jnp.where(b == s, g, _NEG_INF)
        scene_acc[s:s + 1, :] = jnp.maximum(scene_acc[s:s + 1, :],
                                            jnp.max(gs, axis=0, keepdims=True))

    @pl.when(i == pl.num_programs(0) - 1)
    def _():
        scene_out[...] = scene_acc[...]


def _run_sa2(h1g, x2, vmask2, pos, bat2, p2, p3, t=64):
    n = pos.shape[0]
    (w1, b1), (w2, b2), (w3, b3) = p2
    (gw1, gb1), (g2w, g2b), (g3w, g3b) = p3
    w1a, w1b = w1[:128], w1[128:]
    ga, gb = gw1[:256], gw1[256:]
    full = lambda a: pl.BlockSpec(a.shape, lambda i: tuple(0 for _ in a.shape))
    args = (h1g, x2, vmask2, pos, bat2,
            w1a, w1b, b1.reshape(1, -1), w2, b2.reshape(1, -1),
            w3, b3.reshape(1, -1),
            ga, gb, gb1.reshape(1, -1), g2w, g2b.reshape(1, -1),
            g3w, g3b.reshape(1, -1))
    return pl.pallas_call(
        _sa2_body,
        grid=(n // t,),
        in_specs=[pl.BlockSpec((t * _MAXNB, 128), lambda i: (i, 0)),
                  pl.BlockSpec((t * _MAXNB, 3), lambda i: (i, 0)),
                  pl.BlockSpec((t * _MAXNB, 1), lambda i: (i, 0)),
                  pl.BlockSpec((t, 3), lambda i: (i, 0)),
                  pl.BlockSpec((t, 1), lambda i: (i, 0))] +
                 [full(a) for a in args[5:]],
        out_specs=[pl.BlockSpec((t, 256), lambda i: (i, 0)),
                   pl.BlockSpec((_NUM_GRAPHS, 1024), lambda i: (0, 0))],
        out_shape=[jax.ShapeDtypeStruct((n, 256), jnp.float32),
                   jax.ShapeDtypeStruct((_NUM_GRAPHS, 1024), jnp.float32)],
        scratch_shapes=[pltpu.VMEM((_NUM_GRAPHS, 1024), jnp.float32)],
    )(*args)


def _edge_body(h2, scene, bat, out):
    t = h2.shape[0]
    d2 = h2.shape[1]
    b = bat[...]
    e = jnp.broadcast_to(scene[0:1, :], (t, scene.shape[1]))
    for s in range(1, _NUM_GRAPHS):
        e = jnp.where(b == s, scene[s:s + 1, :], e)
    out[:, :d2] = h2[...]
    out[:, d2:] = e


def _run_edge(h2, scene, bat2, t=256):
    n = h2.shape[0]
    d2 = h2.shape[1]
    dtot = d2 + scene.shape[1]
    return pl.pallas_call(
        _edge_body,
        grid=(n // t,),
        in_specs=[pl.BlockSpec((t, d2), lambda i: (i, 0)),
                  pl.BlockSpec(scene.shape, lambda i: (0, 0)),
                  pl.BlockSpec((t, 1), lambda i: (i, 0))],
        out_specs=pl.BlockSpec((t, dtot), lambda i: (i, 0)),
        out_shape=jax.ShapeDtypeStruct((n, dtot), jnp.float32),
    )(h2, scene, bat2)


def kernel(x, pos, batch, params):
    n = pos.shape[0]
    px, py, pz = (pos[:, k] + jnp.float32(0.0) for k in range(3))
    sq = jnp.sum(pos * pos, axis=1)
    bi = batch.astype(jnp.int32)
    starts = jnp.searchsorted(bi, jnp.arange(_NUM_GRAPHS, dtype=jnp.int32))
    ends = jnp.concatenate([starts[1:], jnp.array([n], dtype=starts.dtype)])
    rstart = starts[bi].astype(jnp.int32)
    rend = ends[bi].astype(jnp.int32)

    sel = _make_select_kernel(n)
    r1x, r1y, r1z, r2x, r2y, r2z, nb2, cnt1, cnt2 = sel(
        px, py, pz, sq, rstart, rend)
    r1x, r1y, r1z, r2x, r2y, r2z = (a.reshape(n, _NBUF)
                                    for a in (r1x, r1y, r1z, r2x, r2y, r2z))
    nb2 = nb2.reshape(n, _NBUF)

    slot = jnp.arange(_MAXNB, dtype=jnp.int32)[None, :]
    x1 = jnp.stack([r1x[:, :_MAXNB], r1y[:, :_MAXNB], r1z[:, :_MAXNB]],
                   axis=-1).reshape(n * _MAXNB, 3)
    vmask1 = (slot < cnt1[:, None]).astype(jnp.float32).reshape(n * _MAXNB, 1)
    x2 = jnp.stack([r2x[:, :_MAXNB], r2y[:, :_MAXNB], r2z[:, :_MAXNB]],
                   axis=-1).reshape(n * _MAXNB, 3)
    vmask2 = (slot < cnt2[:, None]).astype(jnp.float32).reshape(n * _MAXNB, 1)

    h1 = _run_sa1(x1, vmask1, params["sa1"])

    idx = nb2[:, :_MAXNB].reshape(n * _MAXNB)
    h1g = _make_gather_kernel(n * _MAXNB, 128)(h1, idx)

    h2, scene = _run_sa2(h1g, x2, vmask2, pos, bi[:, None],
                         params["sa2"], params["sa3"])

    edge = _run_edge(h2, scene, bi[:, None])
    return (scene, edge, batch)


# speedup vs baseline: 1.0179x; 1.0179x over previous
import jax
import jax.numpy as jnp
import numpy as np
from jax import lax
from jax.experimental import pallas as pl
from jax.experimental.pallas import tpu as pltpu
from jax.experimental.pallas import tpu_sc as plsc

_R1SQ = np.float32(0.2 * 0.2)
_R2SQ = np.float32(0.4 * 0.4)
_MAXNB = 64
_NBUF = 80
_NUM_GRAPHS = 8
_NEG_INF = np.float32(-np.inf)


def _make_select_kernel(n_points):
    info = plsc.get_sparse_core_info()
    nc, ns = info.num_cores, info.num_subcores
    nw = nc * ns
    assert n_points % nw == 0
    per_w = n_points // nw
    grp = 16
    assert per_w % grp == 0

    mesh = plsc.VectorSubcoreMesh(core_axis_name="c", subcore_axis_name="s")

    out_type = (
        [jax.ShapeDtypeStruct((n_points * _NBUF,), jnp.float32) for _ in range(6)]
        + [jax.ShapeDtypeStruct((n_points * _NBUF,), jnp.int32)]
        + [jax.ShapeDtypeStruct((n_points,), jnp.int32) for _ in range(2)]
    )

    scratch = (
        [pltpu.VMEM((n_points,), jnp.float32) for _ in range(7)]
        + [pltpu.VMEM((per_w,), jnp.int32) for _ in range(2)]
        + [pltpu.VMEM((grp * _NBUF,), jnp.float32) for _ in range(6)]
        + [pltpu.VMEM((grp * _NBUF,), jnp.int32)]
        + [pltpu.VMEM((grp,), jnp.int32) for _ in range(2)]
    )

    def body(px_h, py_h, pz_h, sq_h, rs_h, re_h,
             r1x_h, r1y_h, r1z_h, r2x_h, r2y_h, r2z_h, nb2_h, c1_h, c2_h,
             px, py, pz, qx, qy, qz, sq, rs, re,
             s1x, s1y, s1z, s2x, s2y, s2z, snb, sc1, sc2):
        wid = lax.axis_index("c") * ns + lax.axis_index("s")
        base = wid * per_w
        for src, dst in ((px_h, px), (py_h, py), (pz_h, pz), (sq_h, sq)):
            pltpu.sync_copy(src, dst)
        pltpu.sync_copy(rs_h.at[pl.ds(base, per_w)], rs)
        pltpu.sync_copy(re_h.at[pl.ds(base, per_w)], re)

        def bf16_round(v):
            u = plsc.bitcast(v, jnp.uint32)
            rb = (u >> jnp.uint32(16)) & jnp.uint32(1)
            u2 = (u + jnp.uint32(0x7FFF) + rb) & jnp.uint32(0xFFFF0000)
            return plsc.bitcast(u2, jnp.float32)

        def qinit(t, _):
            o = t * 16
            qx[pl.ds(o, 16)] = bf16_round(px[pl.ds(o, 16)])
            qy[pl.ds(o, 16)] = bf16_round(py[pl.ds(o, 16)])
            qz[pl.ds(o, 16)] = bf16_round(pz[pl.ds(o, 16)])
            return 0

        lax.fori_loop(0, n_points // 16, qinit, 0)

        lanes = lax.iota(jnp.int32, 16)
        zeros16 = jnp.zeros((16,), jnp.int32)

        def do_group(gi, _):
            gbase = gi * grp
            xv = px[pl.ds(base + gbase, grp)]
            yv = py[pl.ds(base + gbase, grp)]
            zv = pz[pl.ds(base + gbase, grp)]
            qxg = qx[pl.ds(base + gbase, grp)]
            qyg = qy[pl.ds(base + gbase, grp)]
            qzg = qz[pl.ds(base + gbase, grp)]
            sqg = sq[pl.ds(base + gbase, grp)]
            rsv = rs[pl.ds(gbase, grp)]
            rev = re[pl.ds(gbase, grp)]
            cv1 = jnp.zeros((grp,), jnp.int32)
            cv2 = jnp.zeros((grp,), jnp.int32)

            for ip in range(grp):
                xi, yi, zi = xv[ip], yv[ip], zv[ip]
                qxi, qyi, qzi = qxg[ip], qyg[ip], qzg[ip]
                sqi = sqg[ip]
                s = rsv[ip]
                e = rev[ip]

                for k in range(_NBUF // 16):
                    snb[pl.ds(ip * _NBUF + k * 16, 16)] = zeros16

                def chunk(carry, ip=ip):
                    jb, c1, c2 = carry
                    qxv = qx[pl.ds(jb, 16)]
                    qyv = qy[pl.ds(jb, 16)]
                    qzv = qz[pl.ds(jb, 16)]
                    sqv = sq[pl.ds(jb, 16)]
                    cross = qxi * qxv + qyi * qyv + qzi * qzv
                    d2 = (sqi + sqv) - jnp.float32(2.0) * cross
                    inb = lanes < (e - jb)
                    m1 = (d2 <= _R1SQ) & inb
                    m2 = (d2 <= _R2SQ) & inb
                    rx = px[pl.ds(jb, 16)] - xi
                    ry = py[pl.ds(jb, 16)] - yi
                    rz = pz[pl.ds(jb, 16)] - zi
                    jv = jb + lanes

                    m1s = m1 & (c1 < _MAXNB)
                    m2s = m2 & (c2 < _MAXNB)
                    off1 = ip * _NBUF + jnp.minimum(c1, _NBUF - 16)
                    off2 = ip * _NBUF + jnp.minimum(c2, _NBUF - 16)
                    plsc.store_compressed(s1x.at[pl.ds(off1, 16)], rx, mask=m1s)
                    plsc.store_compressed(s1y.at[pl.ds(off1, 16)], ry, mask=m1s)
                    plsc.store_compressed(s1z.at[pl.ds(off1, 16)], rz, mask=m1s)
                    plsc.store_compressed(s2x.at[pl.ds(off2, 16)], rx, mask=m2s)
                    plsc.store_compressed(s2y.at[pl.ds(off2, 16)], ry, mask=m2s)
                    plsc.store_compressed(s2z.at[pl.ds(off2, 16)], rz, mask=m2s)
                    plsc.store_compressed(snb.at[pl.ds(off2, 16)], jv, mask=m2s)

                    n1 = plsc.all_reduce_population_count(m1)[0]
                    n2 = plsc.all_reduce_population_count(m2)[0]
                    c1 = jnp.where(c1 < _MAXNB, c1 + n1, c1)
                    c2 = jnp.where(c2 < _MAXNB, c2 + n2, c2)
                    return jb + 16, c1, c2

                def cond(carry):
                    jb, _, _ = carry
                    return jb < e

                _, c1f, c2f = lax.while_loop(cond, chunk,
                                             (s, jnp.int32(0), jnp.int32(0)))
                lane_is_ip = lanes == ip
                cv1 = jnp.where(lane_is_ip, jnp.minimum(c1f, _MAXNB), cv1)
                cv2 = jnp.where(lane_is_ip, jnp.minimum(c2f, _MAXNB), cv2)

            sc1[...] = cv1
            sc2[...] = cv2

            row = base + gbase
            for st, hb in ((s1x, r1x_h), (s1y, r1y_h), (s1z, r1z_h),
                           (s2x, r2x_h), (s2y, r2y_h), (s2z, r2z_h),
                           (snb, nb2_h)):
                pltpu.sync_copy(st, hb.at[pl.ds(row * _NBUF, grp * _NBUF)])
            pltpu.sync_copy(sc1, c1_h.at[pl.ds(row, grp)])
            pltpu.sync_copy(sc2, c2_h.at[pl.ds(row, grp)])
            return 0

        lax.fori_loop(0, per_w // grp, do_group, 0)

    return pl.kernel(
        body, out_type=out_type, mesh=mesh,
        compiler_params=pltpu.CompilerParams(needs_layout_passes=False),
        scratch_types=scratch)


def _make_gather_kernel(n_rows, d):
    info = plsc.get_sparse_core_info()
    nw = info.num_cores * info.num_subcores
    assert n_rows % nw == 0
    per_w = n_rows // nw
    chunk = 128
    assert per_w % chunk == 0

    mesh = plsc.VectorSubcoreMesh(core_axis_name="c", subcore_axis_name="s")

    def body(tab_h, idx_h, out_h, idx_v, rows_v, sem, osem):
        wid = lax.axis_index("c") * info.num_subcores + lax.axis_index("s")
        base = wid * per_w
        pltpu.sync_copy(idx_h.at[pl.ds(base, per_w)], idx_v)

        def step(t, _):
            pltpu.async_copy(tab_h.at[idx_v.at[pl.ds(t * chunk, chunk)]],
                             rows_v, sem).wait()
            pltpu.async_copy(rows_v,
                             out_h.at[pl.ds(base + t * chunk, chunk), :],
                             osem).wait()
            return 0

        lax.fori_loop(0, per_w // chunk, step, 0)

    return pl.kernel(
        body,
        out_type=jax.ShapeDtypeStruct((n_rows, d), jnp.float32),
        mesh=mesh,
        compiler_params=pltpu.CompilerParams(needs_layout_passes=False),
        scratch_types=[
            pltpu.VMEM((per_w,), jnp.int32),
            pltpu.VMEM((chunk, d), jnp.float32),
            pltpu.SemaphoreType.DMA,
            pltpu.SemaphoreType.DMA,
        ],
    )


def _sa1_body(x, vmask, w1, b1, w2, b2, w3, b3, out):
    rows = x.shape[0]
    t = rows // _MAXNB
    m = jnp.dot(x[...], w1[...], preferred_element_type=jnp.float32)
    m = jnp.maximum(m + b1[...], 0.0)
    m = jnp.dot(m, w2[...], preferred_element_type=jnp.float32)
    m = jnp.maximum(m + b2[...], 0.0)
    m = jnp.dot(m, w3[...], preferred_element_type=jnp.float32)
    m = jnp.maximum(m + b3[...], 0.0)
    m = jnp.where(vmask[...] > 0.5, m, _NEG_INF)
    out[...] = jnp.max(m.reshape(t, _MAXNB, m.shape[-1]), axis=1)


def _run_sa1(x1, vmask1, params, t=128):
    nrows = x1.shape[0]
    n = nrows // _MAXNB
    (w1, b1), (w2, b2), (w3, b3) = params
    fo = w3.shape[1]
    full = lambda a: pl.BlockSpec(a.shape, lambda i: tuple(0 for _ in a.shape))
    args = (x1, vmask1, w1, b1.reshape(1, -1), w2, b2.reshape(1, -1),
            w3, b3.reshape(1, -1))
    return pl.pallas_call(
        _sa1_body,
        grid=(n // t,),
        in_specs=[pl.BlockSpec((t * _MAXNB, 3), lambda i: (i, 0)),
                  pl.BlockSpec((t * _MAXNB, 1), lambda i: (i, 0))] +
                 [full(a) for a in args[2:]],
        out_specs=pl.BlockSpec((t, fo), lambda i: (i, 0)),
        out_shape=jax.ShapeDtypeStruct((n, fo), jnp.float32),
    )(*args)


def _sa2_body(h1g, x2, vmask, pos, bat,
              w1a, w1b, b1, w2, b2, w3, b3,
              ga, gb, gbias, g2w, g2b, g3w, g3b,
              h2_out, scene_out, scene_acc):
    t = pos.shape[0]
    i = pl.program_id(0)

    @pl.when(i == 0)
    def _():
        scene_acc[...] = jnp.full_like(scene_acc[...], _NEG_INF)

    m = jnp.dot(h1g[...], w1a[...], preferred_element_type=jnp.float32)
    m = m + jnp.dot(x2[...], w1b[...], preferred_element_type=jnp.float32)
    m = jnp.maximum(m + b1[...], 0.0)
    m = jnp.dot(m, w2[...], preferred_element_type=jnp.float32)
    m = jnp.maximum(m + b2[...], 0.0)
    m = jnp.dot(m, w3[...], preferred_element_type=jnp.float32)
    m = jnp.maximum(m + b3[...], 0.0)
    m = jnp.where(vmask[...] > 0.5, m, _NEG_INF)
    h2 = jnp.max(m.reshape(t, _MAXNB, m.shape[-1]), axis=1)
    h2_out[...] = h2

    g = jnp.dot(h2, ga[...], preferred_element_type=jnp.float32)
    g = g + jnp.dot(pos[...], gb[...], preferred_element_type=jnp.float32)
    g = jnp.maximum(g + gbias[...], 0.0)
    g = jnp.dot(g, g2w[...], preferred_element_type=jnp.float32)
    g = jnp.maximum(g + g2b[...], 0.0)
    g = jnp.dot(g, g3w[...], preferred_element_type=jnp.float32)
    g = jnp.maximum(g + g3b[...], 0.0)

    b = bat[...]
    for s in range(_NUM_GRAPHS):
        gs = jnp.where(b == s, g, _NEG_INF)
        scene_acc[s:s + 1, :] = jnp.maximum(scene_acc[s:s + 1, :],
                                            jnp.max(gs, axis=0, keepdims=True))

    @pl.when(i == pl.num_programs(0) - 1)
    def _():
        scene_out[...] = scene_acc[...]


def _run_sa2(h1g, x2, vmask2, pos, bat2, p2, p3, t=64):
    n = pos.shape[0]
    (w1, b1), (w2, b2), (w3, b3) = p2
    (gw1, gb1), (g2w, g2b), (g3w, g3b) = p3
    w1a, w1b = w1[:128], w1[128:]
    ga, gb = gw1[:256], gw1[256:]
    full = lambda a: pl.BlockSpec(a.shape, lambda i: tuple(0 for _ in a.shape))
    args = (h1g, x2, vmask2, pos, bat2,
            w1a, w1b, b1.reshape(1, -1), w2, b2.reshape(1, -1),
            w3, b3.reshape(1, -1),
            ga, gb, gb1.reshape(1, -1), g2w, g2b.reshape(1, -1),
            g3w, g3b.reshape(1, -1))
    return pl.pallas_call(
        _sa2_body,
        grid=(n // t,),
        in_specs=[pl.BlockSpec((t * _MAXNB, 128), lambda i: (i, 0)),
                  pl.BlockSpec((t * _MAXNB, 3), lambda i: (i, 0)),
                  pl.BlockSpec((t * _MAXNB, 1), lambda i: (i, 0)),
                  pl.BlockSpec((t, 3), lambda i: (i, 0)),
                  pl.BlockSpec((t, 1), lambda i: (i, 0))] +
                 [full(a) for a in args[5:]],
        out_specs=[pl.BlockSpec((t, 256), lambda i: (i, 0)),
                   pl.BlockSpec((_NUM_GRAPHS, 1024), lambda i: (0, 0))],
        out_shape=[jax.ShapeDtypeStruct((n, 256), jnp.float32),
                   jax.ShapeDtypeStruct((_NUM_GRAPHS, 1024), jnp.float32)],
        scratch_shapes=[pltpu.VMEM((_NUM_GRAPHS, 1024), jnp.float32)],
    )(*args)


def _edge_body(h2, scene, bat, out):
    t = h2.shape[0]
    d2 = h2.shape[1]
    b = bat[...]
    e = jnp.broadcast_to(scene[0:1, :], (t, scene.shape[1]))
    for s in range(1, _NUM_GRAPHS):
        e = jnp.where(b == s, scene[s:s + 1, :], e)
    out[:, :d2] = h2[...]
    out[:, d2:] = e


def _run_edge(h2, scene, bat2, t=256):
    n = h2.shape[0]
    d2 = h2.shape[1]
    dtot = d2 + scene.shape[1]
    return pl.pallas_call(
        _edge_body,
        grid=(n // t,),
        in_specs=[pl.BlockSpec((t, d2), lambda i: (i, 0)),
                  pl.BlockSpec(scene.shape, lambda i: (0, 0)),
                  pl.BlockSpec((t, 1), lambda i: (i, 0))],
        out_specs=pl.BlockSpec((t, dtot), lambda i: (i, 0)),
        out_shape=jax.ShapeDtypeStruct((n, dtot), jnp.float32),
    )(h2, scene, bat2)


def kernel(x, pos, batch, params):
    n = pos.shape[0]
    px, py, pz = (pos[:, k] + jnp.float32(0.0) for k in range(3))
    sq = jnp.sum(pos * pos, axis=1)
    bi = batch.astype(jnp.int32)
    starts = jnp.searchsorted(bi, jnp.arange(_NUM_GRAPHS, dtype=jnp.int32))
    ends = jnp.concatenate([starts[1:], jnp.array([n], dtype=starts.dtype)])
    rstart = starts[bi].astype(jnp.int32)
    rend = ends[bi].astype(jnp.int32)

    sel = _make_select_kernel(n)
    r1x, r1y, r1z, r2x, r2y, r2z, nb2, cnt1, cnt2 = sel(
        px, py, pz, sq, rstart, rend)
    r1x, r1y, r1z, r2x, r2y, r2z = (a.reshape(n, _NBUF)
                                    for a in (r1x, r1y, r1z, r2x, r2y, r2z))
    nb2 = nb2.reshape(n, _NBUF)

    slot = jnp.arange(_MAXNB, dtype=jnp.int32)[None, :]
    x1 = jnp.stack([r1x[:, :_MAXNB], r1y[:, :_MAXNB], r1z[:, :_MAXNB]],
                   axis=-1).reshape(n * _MAXNB, 3)
    vmask1 = (slot < cnt1[:, None]).astype(jnp.float32).reshape(n * _MAXNB, 1)
    x2 = jnp.stack([r2x[:, :_MAXNB], r2y[:, :_MAXNB], r2z[:, :_MAXNB]],
                   axis=-1).reshape(n * _MAXNB, 3)
    vmask2 = (slot < cnt2[:, None]).astype(jnp.float32).reshape(n * _MAXNB, 1)

    h1 = _run_sa1(x1, vmask1, params["sa1"])

    idx = nb2[:, :_MAXNB].reshape(n * _MAXNB)
    h1g = _make_gather_kernel(n * _MAXNB, 128)(h1, idx)

    h2, scene = _run_sa2(h1g, x2, vmask2, pos, bi[:, None],
                         params["sa2"], params["sa3"])

    edge = _run_edge(h2, scene, bi[:, None])
    return (scene, edge, batch)
